# baseline (speedup 1.0000x reference)
import math

import jax
import jax.numpy as jnp
import numpy as np
from jax.experimental import pallas as pl
from jax.experimental.pallas import tpu as pltpu

D = 512
H = 8
L = 6
DFF = 2048
V = 32000
B = 2
S = 1024
DH = D // H
PAD_ID = 0
EMB_SCALE = math.sqrt(D)
NEG = -1e9

QC = 256
BT = 256
NT = 3200


def _posenc(s, d):
    pos = np.arange(s)[:, None].astype(np.float32)
    i = np.arange(0, d, 2)[None, :].astype(np.float32)
    ang = pos / (10000.0 ** (i / d))
    pe = np.zeros((s, d), np.float32)
    pe[:, 0::2] = np.sin(ang)
    pe[:, 1::2] = np.cos(ang)
    return pe


_PE = _posenc(S, D)


def _ln(y, s, b):
    mu = jnp.mean(y, axis=-1, keepdims=True)
    d = y - mu
    var = jnp.mean(d * d, axis=-1, keepdims=True)
    return d * jax.lax.rsqrt(var + 1e-5) * s + b


def _dot(a, w):
    return jnp.dot(a, w, preferred_element_type=jnp.float32)


def _embed_body(ids_ref, emb_hbm, pe_ref, out_ref, buf, sem):
    i = pl.program_id(0)
    base = i * BT
    copies = []
    for t in range(BT):
        idx = ids_ref[base + t]
        cp = pltpu.make_async_copy(emb_hbm.at[idx], buf.at[t], sem)
        cp.start()
        copies.append(cp)
    for cp in copies:
        cp.wait()
    out_ref[...] = buf[...] * EMB_SCALE + pe_ref[...]


def _embed(ids_flat, emb):
    n = ids_flat.shape[0]
    pe_blocks = S // BT
    return pl.pallas_call(
        _embed_body,
        out_shape=jax.ShapeDtypeStruct((n, D), jnp.float32),
        grid_spec=pltpu.PrefetchScalarGridSpec(
            num_scalar_prefetch=1,
            grid=(n // BT,),
            in_specs=[
                pl.BlockSpec(memory_space=pl.ANY),
                pl.BlockSpec((BT, D), lambda i, ids: (i % pe_blocks, 0)),
            ],
            out_specs=pl.BlockSpec((BT, D), lambda i, ids: (i, 0)),
            scratch_shapes=[
                pltpu.VMEM((BT, D), jnp.float32),
                pltpu.SemaphoreType.DMA,
            ],
        ),
        compiler_params=pltpu.CompilerParams(
            dimension_semantics=("arbitrary",),
        ),
        name="embed_gather",
    )(ids_flat, emb, jnp.asarray(_PE))


def _attn_chunks(q_ref, q_off, kv_ref, k_off, v_off, padf, causal, o_scr):
    scale = DH ** -0.5
    for r0 in range(0, S, QC):
        if causal:
            rows = jax.lax.broadcasted_iota(jnp.int32, (QC, S), 0) + r0
            cols = jax.lax.broadcasted_iota(jnp.int32, (QC, S), 1)
            cmask = cols > rows
        for h in range(H):
            q = q_ref[r0:r0 + QC, q_off + h * DH:q_off + (h + 1) * DH]
            k = kv_ref[:, k_off + h * DH:k_off + (h + 1) * DH]
            v = kv_ref[:, v_off + h * DH:v_off + (h + 1) * DH]
            sc = jax.lax.dot_general(
                q, k, (((1,), (1,)), ((), ())),
                preferred_element_type=jnp.float32) * scale
            if causal:
                sc = jnp.where(cmask, NEG, sc)
            else:
                sc = jnp.where(padf > 0.5, NEG, sc)
            m = jnp.max(sc, axis=-1, keepdims=True)
            p = jnp.exp(sc - m)
            l = jnp.sum(p, axis=-1, keepdims=True)
            p = p / l
            o_scr[r0:r0 + QC, h * DH:(h + 1) * DH] = _dot(p, v)


def _proj_resid_ln(src_scr, w_ref, state, lns, lnb):
    for r0 in range(0, S, QC):
        proj = _dot(src_scr[r0:r0 + QC, :], w_ref[...])
        state[r0:r0 + QC, :] = _ln(state[r0:r0 + QC, :] + proj, lns, lnb)


def _ffn_chunks(state, w1_ref, fb1, w2_ref, fb2, lns, lnb, out_ref, h_scr):
    for r0 in range(0, S, QC):
        xc = state[r0:r0 + QC, :]
        h_scr[...] = jnp.maximum(_dot(xc, w1_ref[...]) + fb1, 0.0)
        y = _dot(h_scr[...], w2_ref[...]) + fb2 + xc
        yo = _ln(y, lns, lnb)
        state[r0:r0 + QC, :] = yo
        out_ref[0, r0:r0 + QC, :] = yo


def _enc_body(x_ref, wqkv_ref, wo_ref, l1s_ref, l1b_ref, w1_ref, fb1_ref,
              w2_ref, fb2_ref, l2s_ref, l2b_ref, padf_ref, out_ref,
              xs, qkv, o_scr, h_scr):
    lyr = pl.program_id(1)

    @pl.when(lyr == 0)
    def _():
        xs[...] = x_ref[0]

    padf = padf_ref[0]
    qkv[...] = _dot(xs[...], wqkv_ref[0])
    _attn_chunks(qkv, 0, qkv, D, 2 * D, padf, False, o_scr)
    _proj_resid_ln(o_scr, wo_ref.at[0], xs, l1s_ref[0], l1b_ref[0])
    _ffn_chunks(xs, w1_ref.at[0], fb1_ref[0], w2_ref.at[0], fb2_ref[0],
                l2s_ref[0], l2b_ref[0], out_ref, h_scr)


def _enc_layers(x, wqkv, wo, l1s, l1b, w1, fb1, w2, fb2, l2s, l2b, padf):
    return pl.pallas_call(
        _enc_body,
        out_shape=jax.ShapeDtypeStruct((B, S, D), jnp.float32),
        grid=(B, L),
        in_specs=[
            pl.BlockSpec((1, S, D), lambda b, l: (b, 0, 0)),
            pl.BlockSpec((1, D, 3 * D), lambda b, l: (l, 0, 0)),
            pl.BlockSpec((1, D, D), lambda b, l: (l, 0, 0)),
            pl.BlockSpec((1, 1, D), lambda b, l: (l, 0, 0)),
            pl.BlockSpec((1, 1, D), lambda b, l: (l, 0, 0)),
            pl.BlockSpec((1, D, DFF), lambda b, l: (l, 0, 0)),
            pl.BlockSpec((1, 1, DFF), lambda b, l: (l, 0, 0)),
            pl.BlockSpec((1, DFF, D), lambda b, l: (l, 0, 0)),
            pl.BlockSpec((1, 1, D), lambda b, l: (l, 0, 0)),
            pl.BlockSpec((1, 1, D), lambda b, l: (l, 0, 0)),
            pl.BlockSpec((1, 1, D), lambda b, l: (l, 0, 0)),
            pl.BlockSpec((1, 1, S), lambda b, l: (b, 0, 0)),
        ],
        out_specs=pl.BlockSpec((1, S, D), lambda b, l: (b, 0, 0)),
        scratch_shapes=[
            pltpu.VMEM((S, D), jnp.float32),
            pltpu.VMEM((S, 3 * D), jnp.float32),
            pltpu.VMEM((S, D), jnp.float32),
            pltpu.VMEM((QC, DFF), jnp.float32),
        ],
        compiler_params=pltpu.CompilerParams(
            dimension_semantics=("arbitrary", "arbitrary"),
            vmem_limit_bytes=58 * 1024 * 1024,
        ),
        name="enc_layers",
    )(x, wqkv, wo, l1s, l1b, w1, fb1, w2, fb2, l2s, l2b, padf)


def _dec_body(y_ref, enc_ref, wqkv_ref, wo_ref, l1s_ref, l1b_ref,
              wq_ref, wkv_ref, woc_ref, l2s_ref, l2b_ref,
              w1_ref, fb1_ref, w2_ref, fb2_ref, l3s_ref, l3b_ref,
              padf_ref, out_ref, ys, qkv, o_scr, h_scr):
    lyr = pl.program_id(1)

    @pl.when(lyr == 0)
    def _():
        ys[...] = y_ref[0]

    padf = padf_ref[0]

    qkv[...] = _dot(ys[...], wqkv_ref[0])
    _attn_chunks(qkv, 0, qkv, D, 2 * D, None, True, o_scr)
    _proj_resid_ln(o_scr, wo_ref.at[0], ys, l1s_ref[0], l1b_ref[0])

    qkv[:, 0:2 * D] = _dot(enc_ref[0], wkv_ref[0])
    qkv[:, 2 * D:3 * D] = _dot(ys[...], wq_ref[0])
    _attn_chunks(qkv, 2 * D, qkv, 0, D, padf, False, o_scr)
    _proj_resid_ln(o_scr, woc_ref.at[0], ys, l2s_ref[0], l2b_ref[0])

    _ffn_chunks(ys, w1_ref.at[0], fb1_ref[0], w2_ref.at[0], fb2_ref[0],
                l3s_ref[0], l3b_ref[0], out_ref, h_scr)


def _dec_layers(y, enc_out, wqkv, wo, l1s, l1b, wq, wkv, woc, l2s, l2b,
                w1, fb1, w2, fb2, l3s, l3b, padf):
    return pl.pallas_call(
        _dec_body,
        out_shape=jax.ShapeDtypeStruct((B, S, D), jnp.float32),
        grid=(B, L),
        in_specs=[
            pl.BlockSpec((1, S, D), lambda b, l: (b, 0, 0)),
            pl.BlockSpec((1, S, D), lambda b, l: (b, 0, 0)),
            pl.BlockSpec((1, D, 3 * D), lambda b, l: (l, 0, 0)),
            pl.BlockSpec((1, D, D), lambda b, l: (l, 0, 0)),
            pl.BlockSpec((1, 1, D), lambda b, l: (l, 0, 0)),
            pl.BlockSpec((1, 1, D), lambda b, l: (l, 0, 0)),
            pl.BlockSpec((1, D, D), lambda b, l: (l, 0, 0)),
            pl.BlockSpec((1, D, 2 * D), lambda b, l: (l, 0, 0)),
            pl.BlockSpec((1, D, D), lambda b, l: (l, 0, 0)),
            pl.BlockSpec((1, 1, D), lambda b, l: (l, 0, 0)),
            pl.BlockSpec((1, 1, D), lambda b, l: (l, 0, 0)),
            pl.BlockSpec((1, D, DFF), lambda b, l: (l, 0, 0)),
            pl.BlockSpec((1, 1, DFF), lambda b, l: (l, 0, 0)),
            pl.BlockSpec((1, DFF, D), lambda b, l: (l, 0, 0)),
            pl.BlockSpec((1, 1, D), lambda b, l: (l, 0, 0)),
            pl.BlockSpec((1, 1, D), lambda b, l: (l, 0, 0)),
            pl.BlockSpec((1, 1, D), lambda b, l: (l, 0, 0)),
            pl.BlockSpec((1, 1, S), lambda b, l: (b, 0, 0)),
        ],
        out_specs=pl.BlockSpec((1, S, D), lambda b, l: (b, 0, 0)),
        scratch_shapes=[
            pltpu.VMEM((S, D), jnp.float32),
            pltpu.VMEM((S, 3 * D), jnp.float32),
            pltpu.VMEM((S, D), jnp.float32),
            pltpu.VMEM((QC, DFF), jnp.float32),
        ],
        compiler_params=pltpu.CompilerParams(
            dimension_semantics=("arbitrary", "arbitrary"),
            vmem_limit_bytes=58 * 1024 * 1024,
        ),
        name="dec_layers",
    )(y, enc_out, wqkv, wo, l1s, l1b, wq, wkv, woc, l2s, l2b,
      w1, fb1, w2, fb2, l3s, l3b, padf)


def _logits_body(x_ref, w_ref, b_ref, out_ref):
    out_ref[...] = (jnp.dot(x_ref[...], w_ref[...],
                            preferred_element_type=jnp.float32)
                    + b_ref[...])


def _logits(x2d, fc_w, fc_b):
    n = x2d.shape[0]
    mt = n // 2
    return pl.pallas_call(
        _logits_body,
        out_shape=jax.ShapeDtypeStruct((n, V), jnp.float32),
        grid=(V // NT, 2),
        in_specs=[
            pl.BlockSpec((mt, D), lambda j, i: (i, 0)),
            pl.BlockSpec((D, NT), lambda j, i: (0, j)),
            pl.BlockSpec((1, NT), lambda j, i: (0, j)),
        ],
        out_specs=pl.BlockSpec((mt, NT), lambda j, i: (i, j)),
        compiler_params=pltpu.CompilerParams(
            dimension_semantics=("parallel", "arbitrary"),
            vmem_limit_bytes=56 * 1024 * 1024,
        ),
        name="vocab_proj",
    )(x2d, fc_w, fc_b.reshape(1, V))


def kernel(batch_src, trg_teacher, src_emb, trg_emb, fc_w, fc_b,
           enc_wqkv, enc_wo, enc_ln1s, enc_ln1b, enc_w1, enc_b1, enc_w2,
           enc_b2, enc_ln2s, enc_ln2b,
           dec_wqkv, dec_wo, dec_ln1s, dec_ln1b, dec_wq, dec_wkv, dec_woc,
           dec_ln2s, dec_ln2b, dec_w1, dec_b1, dec_w2, dec_b2, dec_ln3s,
           dec_ln3b):
    padf = (batch_src == PAD_ID).astype(jnp.float32).reshape(B, 1, S)
    r3 = lambda a: a.reshape(L, 1, -1)

    x = _embed(batch_src.reshape(-1), src_emb).reshape(B, S, D)
    enc_out = _enc_layers(x, enc_wqkv, enc_wo, r3(enc_ln1s), r3(enc_ln1b),
                          enc_w1, r3(enc_b1), enc_w2, r3(enc_b2),
                          r3(enc_ln2s), r3(enc_ln2b), padf)

    y = _embed(trg_teacher.reshape(-1), trg_emb).reshape(B, S, D)
    y = _dec_layers(y, enc_out, dec_wqkv, dec_wo, r3(dec_ln1s), r3(dec_ln1b),
                    dec_wq, dec_wkv, dec_woc, r3(dec_ln2s), r3(dec_ln2b),
                    dec_w1, r3(dec_b1), dec_w2, r3(dec_b2), r3(dec_ln3s),
                    r3(dec_ln3b), padf)

    return _logits(y.reshape(B * S, D), fc_w, fc_b).reshape(B, S, V)
